# Initial kernel scaffold; baseline (speedup 1.0000x reference)
import jax
import jax.numpy as jnp
from jax.experimental import pallas as pl


def kernel(x, edge_index, params):
    raise NotImplementedError("write your pallas kernel here")


# speedup vs baseline: 1.0049x; 1.0049x over previous
import functools

import jax
import jax.numpy as jnp
from jax.experimental import pallas as pl
from jax.experimental.pallas import tpu as pltpu

_DIMS = [(128, 64), (64, 64), (64, 64), (64, 128), (128, 1024), (1024, 512), (512, 256), (256, 40)]
_N = 10000


def _matmul_kern(h_ref, w_ref, o_ref):
    o_ref[...] = jnp.dot(h_ref[...], w_ref[...], preferred_element_type=jnp.float32)


def _matmul(h, W, block=1000):
    n, din = h.shape
    dout = W.shape[1]
    return pl.pallas_call(
        _matmul_kern,
        grid=(n // block,),
        in_specs=[
            pl.BlockSpec((block, din), lambda i: (i, 0)),
            pl.BlockSpec((din, dout), lambda i: (0, 0)),
        ],
        out_specs=pl.BlockSpec((block, dout), lambda i: (i, 0)),
        out_shape=jax.ShapeDtypeStruct((n, dout), jnp.float32),
        compiler_params=pltpu.CompilerParams(
            dimension_semantics=("parallel",),
        ),
    )(h, W)


def kernel(x, edge_index, params):
    n_nodes = x.shape[0]
    loops = jnp.arange(n_nodes, dtype=edge_index.dtype)
    src = jnp.concatenate([edge_index[0], loops])
    dst = jnp.concatenate([edge_index[1], loops])
    deg = jax.ops.segment_sum(jnp.ones(src.shape[0], jnp.float32), dst, num_segments=n_nodes)
    deg_safe = jnp.where(deg > 0, deg, 1.0)
    dis = jnp.where(deg > 0, 1.0 / jnp.sqrt(deg_safe), 0.0)
    norm = dis[src] * dis[dst]

    h = x
    for i in range(len(_DIMS)):
        W = params[f"W{i}"]
        b = params[f"b{i}"]
        hw = _matmul(h, W)
        msg = jnp.take(hw, src, axis=0) * norm[:, None]
        agg = jax.ops.segment_sum(msg, dst, num_segments=n_nodes) + b
        mean = agg.mean(axis=0)
        var = agg.var(axis=0)
        y = (agg - mean) / jnp.sqrt(var + 1e-5) * params[f"g{i}"] + params[f"be{i}"]
        h = jax.nn.relu(y)
    return h
